# Initial kernel scaffold; baseline (speedup 1.0000x reference)
import functools

import jax
import jax.numpy as jnp
from jax import lax
from jax.experimental import pallas as pl
from jax.experimental.pallas import tpu as pltpu
from jax.experimental.pallas import tpu_sc as plsc

K = 32
DPAD = 272


def _lrelu(x):
    return jnp.where(x > 0, x, 0.1 * x)


def _p1conv_body(x_ref, w1, b1, g1, e1, w2, b2, g2, e2, w3, b3, g3, e3, o_ref):
    def layer(x, wt, bt, g, e):
        y = jnp.dot(x, wt[...], preferred_element_type=jnp.float32) + bt[...]
        m = jnp.mean(y, axis=0, keepdims=True)
        v = jnp.mean((y - m) ** 2, axis=0, keepdims=True)
        return _lrelu(g[...] * (y - m) / jnp.sqrt(v + 1e-5) + e[...])

    x = layer(x_ref[...], w1, b1, g1, e1)
    x = layer(x, w2, b2, g2, e2)
    o_ref[...] = layer(x, w3, b3, g3, e3)


def _p1conv(xp, w1t, b1p, g1, e1, w2t, b2p, g2, e2, w3t, b3p, g3, e3):
    P, H = xp.shape[0], w3t.shape[1]
    return pl.pallas_call(
        _p1conv_body,
        out_shape=jax.ShapeDtypeStruct((P, H), jnp.float32),
    )(xp, w1t, b1p, g1, e1, w2t, b2p, g2, e2, w3t, b3p, g3, e3)


def _knn_body(x1_ref, x2_ref, o_ref, dsc, n2):
    b = pl.program_id(0)
    x1t = jnp.swapaxes(x1_ref[0], 0, 1)
    x2 = x2_ref[0]
    prod = jnp.dot(x1t, x2, preferred_element_type=jnp.float32)
    sq1 = jnp.sum(x1t * x1t, axis=1, keepdims=True)
    sq2 = jnp.sum(x2 * x2, axis=0, keepdims=True)
    dsc[...] = ((-2.0 * prod) + sq1) + sq2

    n1 = x1t.shape[0]
    lane = lax.broadcasted_iota(jnp.int32, (n1, n2), 1)
    col = lax.broadcasted_iota(jnp.int32, (n1, K), 1)

    def step(k, acc):
        d = dsc[...]
        idx = jnp.argmin(d, axis=1).astype(jnp.int32)[:, None]
        dsc[...] = jnp.where(lane == idx, jnp.float32(1e30), d)
        return jnp.where(col == k, idx, acc)

    acc = lax.fori_loop(0, K, step, jnp.zeros((n1, K), jnp.int32))
    o_ref[0] = acc + b * n2


def _knn(xyz1, xyz2):
    B, _, N1 = xyz1.shape
    N2 = xyz2.shape[2]
    return pl.pallas_call(
        functools.partial(_knn_body, n2=N2),
        grid=(B,),
        in_specs=[
            pl.BlockSpec((1, 3, N1), lambda b: (b, 0, 0)),
            pl.BlockSpec((1, 3, N2), lambda b: (b, 0, 0)),
        ],
        out_specs=pl.BlockSpec((1, N1, K), lambda b: (b, 0, 0)),
        out_shape=jax.ShapeDtypeStruct((B, N1, K), jnp.int32),
        scratch_shapes=[pltpu.VMEM((N1, N2), jnp.float32)],
    )(xyz1, xyz2)


def _table_body(p2_ref, x2_ref, o_ref):
    o_ref[:, 0:256] = jnp.swapaxes(p2_ref[0], 0, 1)
    o_ref[:, 256:259] = jnp.swapaxes(x2_ref[0], 0, 1)
    o_ref[:, 259:DPAD] = jnp.zeros_like(o_ref[:, 259:DPAD])


def _build_table(points2, xyz2):
    B, C, N2 = points2.shape
    J = 8
    W = N2 // J
    return pl.pallas_call(
        _table_body,
        grid=(B, J),
        in_specs=[
            pl.BlockSpec((1, C, W), lambda b, j: (b, 0, j)),
            pl.BlockSpec((1, 3, W), lambda b, j: (b, 0, j)),
        ],
        out_specs=pl.BlockSpec((W, DPAD), lambda b, j: (b * J + j, 0)),
        out_shape=jax.ShapeDtypeStruct((B * N2, DPAD), jnp.float32),
    )(points2, xyz2)


def _sc_gather_rows(table, idx2d):
    nsel = idx2d.shape[1]
    W = 128
    mesh = plsc.VectorSubcoreMesh(core_axis_name="c", subcore_axis_name="s")

    @functools.partial(
        pl.kernel,
        out_type=jax.ShapeDtypeStruct((nsel, DPAD), jnp.float32),
        mesh=mesh,
    )
    def k(table_hbm, i_hbm, o_hbm):
        def body(i_vmem, o_vmem):
            pltpu.sync_copy(table_hbm.at[i_vmem.at[0]], o_vmem)

        pltpu.emit_pipeline(
            body,
            grid=(nsel // W,),
            in_specs=[pl.BlockSpec((1, W), lambda i: (0, i))],
            out_specs=[pl.BlockSpec((W, DPAD), lambda i: (i, 0))],
            core_axis_name=("c", "s"),
            dimension_semantics=(pltpu.PARALLEL,),
        )(i_hbm, o_hbm)

    return k(table, idx2d)


def _mlp_body(g_ref, p1_ref, x1_ref, wat, wbt, wct, bm1, w2t, bm2, wq1t,
              wq2t, wrt, br, o_ref):
    n1 = p1_ref.shape[0]
    nbr = g_ref[:, 256:259]
    x1t = jnp.swapaxes(x1_ref[0], 0, 1)
    x1rep = jnp.broadcast_to(x1t[:, None, :], (n1, K, 3)).reshape(n1 * K, 3)
    dirn = nbr - x1rep

    z1 = jnp.dot(p1_ref[...], wat[...], preferred_element_type=jnp.float32)
    z1rep = jnp.broadcast_to(z1[:, None, :], (n1, K, 512)).reshape(n1 * K, 512)
    zp2 = jnp.dot(g_ref[:, 0:256], wbt[...], preferred_element_type=jnp.float32)
    zdir = jnp.dot(dirn, wct[...], preferred_element_type=jnp.float32)
    h1 = _lrelu(zp2 + zdir + z1rep + bm1[...])
    h2 = _lrelu(jnp.dot(h1, w2t[...], preferred_element_type=jnp.float32)
                + bm2[...])
    pooled = jnp.max(h2.reshape(n1, K, 512), axis=1)
    q = _lrelu(jnp.dot(pooled, wq1t[...], preferred_element_type=jnp.float32))
    q = _lrelu(jnp.dot(q, wq2t[...], preferred_element_type=jnp.float32))
    o_ref[0] = jnp.dot(q, wrt[...], preferred_element_type=jnp.float32) + br[...]


def _mlp(grows, p1, xyz1, wat, wbt, wct, bm1, w2t, bm2, wq1t, wq2t, wrt, br):
    B, _, N1 = xyz1.shape
    H = wat.shape[0]
    return pl.pallas_call(
        _mlp_body,
        grid=(B,),
        in_specs=[
            pl.BlockSpec((N1 * K, DPAD), lambda b: (b, 0)),
            pl.BlockSpec((N1, H), lambda b: (b, 0)),
            pl.BlockSpec((1, 3, N1), lambda b: (b, 0, 0)),
            pl.BlockSpec(wat.shape, lambda b: (0, 0)),
            pl.BlockSpec(wbt.shape, lambda b: (0, 0)),
            pl.BlockSpec(wct.shape, lambda b: (0, 0)),
            pl.BlockSpec((1, 512), lambda b: (0, 0)),
            pl.BlockSpec(w2t.shape, lambda b: (0, 0)),
            pl.BlockSpec((1, 512), lambda b: (0, 0)),
            pl.BlockSpec(wq1t.shape, lambda b: (0, 0)),
            pl.BlockSpec(wq2t.shape, lambda b: (0, 0)),
            pl.BlockSpec(wrt.shape, lambda b: (0, 0)),
            pl.BlockSpec((1, 3), lambda b: (0, 0)),
        ],
        out_specs=pl.BlockSpec((1, N1, 3), lambda b: (b, 0, 0)),
        out_shape=jax.ShapeDtypeStruct((B, N1, 3), jnp.float32),
    )(grows, p1, xyz1, wat, wbt, wct, bm1, w2t, bm2, wq1t, wq2t, wrt, br)


def kernel(xyz1, xyz2, points1, points2, W1, b1, g1, be1, W2, b2, g2, be2,
           W3, b3, g3, be3, Wm1, bm1, Wm2, bm2, Wq1, Wq2, Wr, br):
    B, _, N1 = xyz1.shape
    N2 = xyz2.shape[2]
    LC = points1.shape[1]
    IC = points2.shape[1]
    H = W1.shape[0]

    xp = jnp.transpose(points1, (0, 2, 1)).reshape(B * N1, LC)
    b1p = jnp.tile(jnp.transpose(b1), (B, 1))
    b2p = jnp.tile(jnp.transpose(b2), (B, 1))
    b3p = jnp.tile(jnp.transpose(b3), (B, 1))
    p1 = _p1conv(xp, jnp.transpose(W1), b1p, g1[None, :], be1[None, :],
                 jnp.transpose(W2), b2p, g2[None, :], be2[None, :],
                 jnp.transpose(W3), b3p, g3[None, :], be3[None, :])

    idx = _knn(xyz1, xyz2)
    table = _build_table(points2, xyz2)
    grows = _sc_gather_rows(table, idx.reshape(1, B * N1 * K))

    wat = jnp.transpose(Wm1[:, 0:H])
    wbt = jnp.transpose(Wm1[:, H:H + IC])
    wct = jnp.transpose(Wm1[:, H + IC:H + IC + 3])
    out = _mlp(grows, p1.reshape(B, N1, H), xyz1, wat, wbt, wct,
               bm1[None, :], jnp.transpose(Wm2), bm2[None, :],
               jnp.transpose(Wq1), jnp.transpose(Wq2), jnp.transpose(Wr),
               br[None, :])
    return jnp.transpose(out, (0, 2, 1))


# speedup vs baseline: 5.8351x; 5.8351x over previous
import functools

import jax
import jax.numpy as jnp
from jax import lax
from jax.experimental import pallas as pl
from jax.experimental.pallas import tpu as pltpu
from jax.experimental.pallas import tpu_sc as plsc

K = 32


def _dpad(c):
    return ((c + 3 + 127) // 128) * 128


def _lrelu(x):
    return jnp.where(x > 0, x, 0.1 * x)


def _p1conv_body(x_ref, w1, b1, g1, e1, w2, b2, g2, e2, w3, b3, g3, e3, o_ref):
    def layer(x, wt, bt, g, e):
        y = jnp.dot(x, wt[...], preferred_element_type=jnp.float32) + bt[...]
        m = jnp.mean(y, axis=0, keepdims=True)
        v = jnp.mean((y - m) ** 2, axis=0, keepdims=True)
        return _lrelu(g[...] * (y - m) / jnp.sqrt(v + 1e-5) + e[...])

    x = layer(x_ref[...], w1, b1, g1, e1)
    x = layer(x, w2, b2, g2, e2)
    o_ref[...] = layer(x, w3, b3, g3, e3)


def _p1conv(xp, w1t, b1p, g1, e1, w2t, b2p, g2, e2, w3t, b3p, g3, e3):
    P, H = xp.shape[0], w3t.shape[1]
    return pl.pallas_call(
        _p1conv_body,
        out_shape=jax.ShapeDtypeStruct((P, H), jnp.float32),
    )(xp, w1t, b1p, g1, e1, w2t, b2p, g2, e2, w3t, b3p, g3, e3)


def _knn_body(x1_ref, x2_ref, o_ref, dsc, n2):
    b = pl.program_id(0)
    x1t = jnp.swapaxes(x1_ref[0], 0, 1)
    x2 = x2_ref[0]
    prod = jnp.dot(x1t, x2, preferred_element_type=jnp.float32)
    sq1 = jnp.sum(x1t * x1t, axis=1, keepdims=True)
    sq2 = jnp.sum(x2 * x2, axis=0, keepdims=True)
    dsc[...] = ((-2.0 * prod) + sq1) + sq2

    n1 = x1t.shape[0]
    lane = lax.broadcasted_iota(jnp.int32, (n1, n2), 1)
    col = lax.broadcasted_iota(jnp.int32, (n1, K), 1)

    def step(k, acc):
        d = dsc[...]
        idx = jnp.argmin(d, axis=1).astype(jnp.int32)[:, None]
        dsc[...] = jnp.where(lane == idx, jnp.float32(1e30), d)
        return jnp.where(col == k, idx, acc)

    acc = lax.fori_loop(0, K, step, jnp.zeros((n1, K), jnp.int32))
    o_ref[0] = acc + b * n2


def _knn(xyz1, xyz2):
    B, _, N1 = xyz1.shape
    N2 = xyz2.shape[2]
    return pl.pallas_call(
        functools.partial(_knn_body, n2=N2),
        grid=(B,),
        in_specs=[
            pl.BlockSpec((1, 3, N1), lambda b: (b, 0, 0)),
            pl.BlockSpec((1, 3, N2), lambda b: (b, 0, 0)),
        ],
        out_specs=pl.BlockSpec((1, N1, K), lambda b: (b, 0, 0)),
        out_shape=jax.ShapeDtypeStruct((B, N1, K), jnp.int32),
        scratch_shapes=[pltpu.VMEM((N1, N2), jnp.float32)],
    )(xyz1, xyz2)


def _table_body(p2_ref, x2_ref, o_ref, c, dpad):
    o_ref[:, 0:c] = jnp.swapaxes(p2_ref[0], 0, 1)
    o_ref[:, c:c + 3] = jnp.swapaxes(x2_ref[0], 0, 1)


def _build_table(points2, xyz2):
    B, C, N2 = points2.shape
    dpad = _dpad(C)
    J = 8
    W = N2 // J
    return pl.pallas_call(
        functools.partial(_table_body, c=C, dpad=dpad),
        grid=(B, J),
        in_specs=[
            pl.BlockSpec((1, C, W), lambda b, j: (b, 0, j)),
            pl.BlockSpec((1, 3, W), lambda b, j: (b, 0, j)),
        ],
        out_specs=pl.BlockSpec((W, dpad), lambda b, j: (b * J + j, 0)),
        out_shape=jax.ShapeDtypeStruct((B * N2, dpad), jnp.float32),
    )(points2, xyz2)


def _sc_gather_rows(table, idx2d):
    nsel = idx2d.shape[1]
    dpad = table.shape[1]
    W = 128
    mesh = plsc.VectorSubcoreMesh(core_axis_name="c", subcore_axis_name="s")

    @functools.partial(
        pl.kernel,
        out_type=jax.ShapeDtypeStruct((nsel, dpad), jnp.float32),
        mesh=mesh,
    )
    def k(table_hbm, i_hbm, o_hbm):
        def body(i_vmem, o_vmem):
            pltpu.sync_copy(table_hbm.at[i_vmem.at[0]], o_vmem)

        pltpu.emit_pipeline(
            body,
            grid=(nsel // W,),
            in_specs=[pl.BlockSpec((1, W), lambda i: (0, i))],
            out_specs=[pl.BlockSpec((W, dpad), lambda i: (i, 0))],
            core_axis_name=("c", "s"),
            dimension_semantics=(pltpu.PARALLEL,),
        )(i_hbm, o_hbm)

    return k(table, idx2d)


def _mlp_body(g_ref, p1_ref, x1_ref, wat, wbt, wct, bm1, w2t, bm2, wq1t,
              wq2t, wrt, br, o_ref, c):
    n1 = p1_ref.shape[0]
    h2dim = wat.shape[1]
    nbr = g_ref[:, c:c + 3]
    x1t = jnp.swapaxes(x1_ref[0], 0, 1)
    x1rep = jnp.broadcast_to(x1t[:, None, :], (n1, K, 3)).reshape(n1 * K, 3)
    dirn = nbr - x1rep

    z1 = jnp.dot(p1_ref[...], wat[...], preferred_element_type=jnp.float32)
    z1rep = jnp.broadcast_to(z1[:, None, :],
                             (n1, K, h2dim)).reshape(n1 * K, h2dim)
    zp2 = jnp.dot(g_ref[:, 0:c], wbt[...], preferred_element_type=jnp.float32)
    zdir = jnp.dot(dirn, wct[...], preferred_element_type=jnp.float32)
    h1 = _lrelu(zp2 + zdir + z1rep + bm1[...])
    h2 = _lrelu(jnp.dot(h1, w2t[...], preferred_element_type=jnp.float32)
                + bm2[...])
    pooled = jnp.max(h2.reshape(n1, K, h2dim), axis=1)
    q = _lrelu(jnp.dot(pooled, wq1t[...], preferred_element_type=jnp.float32))
    q = _lrelu(jnp.dot(q, wq2t[...], preferred_element_type=jnp.float32))
    o_ref[0] = jnp.dot(q, wrt[...], preferred_element_type=jnp.float32) + br[...]


def _mlp(grows, p1, xyz1, wat, wbt, wct, bm1, w2t, bm2, wq1t, wq2t, wrt, br):
    B, _, N1 = xyz1.shape
    H = wat.shape[0]
    C = wbt.shape[0]
    dpad = grows.shape[1]
    h2dim = wat.shape[1]
    return pl.pallas_call(
        functools.partial(_mlp_body, c=C),
        grid=(B,),
        in_specs=[
            pl.BlockSpec((N1 * K, dpad), lambda b: (b, 0)),
            pl.BlockSpec((N1, H), lambda b: (b, 0)),
            pl.BlockSpec((1, 3, N1), lambda b: (b, 0, 0)),
            pl.BlockSpec(wat.shape, lambda b: (0, 0)),
            pl.BlockSpec(wbt.shape, lambda b: (0, 0)),
            pl.BlockSpec(wct.shape, lambda b: (0, 0)),
            pl.BlockSpec((1, h2dim), lambda b: (0, 0)),
            pl.BlockSpec(w2t.shape, lambda b: (0, 0)),
            pl.BlockSpec((1, h2dim), lambda b: (0, 0)),
            pl.BlockSpec(wq1t.shape, lambda b: (0, 0)),
            pl.BlockSpec(wq2t.shape, lambda b: (0, 0)),
            pl.BlockSpec(wrt.shape, lambda b: (0, 0)),
            pl.BlockSpec((1, 3), lambda b: (0, 0)),
        ],
        out_specs=pl.BlockSpec((1, N1, 3), lambda b: (b, 0, 0)),
        out_shape=jax.ShapeDtypeStruct((B, N1, 3), jnp.float32),
    )(grows, p1, xyz1, wat, wbt, wct, bm1, w2t, bm2, wq1t, wq2t, wrt, br)


def kernel(xyz1, xyz2, points1, points2, W1, b1, g1, be1, W2, b2, g2, be2,
           W3, b3, g3, be3, Wm1, bm1, Wm2, bm2, Wq1, Wq2, Wr, br):
    B, _, N1 = xyz1.shape
    N2 = xyz2.shape[2]
    LC = points1.shape[1]
    IC = points2.shape[1]
    H = W1.shape[0]

    xp = jnp.transpose(points1, (0, 2, 1)).reshape(B * N1, LC)
    b1p = jnp.tile(jnp.transpose(b1), (B, 1))
    b2p = jnp.tile(jnp.transpose(b2), (B, 1))
    b3p = jnp.tile(jnp.transpose(b3), (B, 1))
    p1 = _p1conv(xp, jnp.transpose(W1), b1p, g1[None, :], be1[None, :],
                 jnp.transpose(W2), b2p, g2[None, :], be2[None, :],
                 jnp.transpose(W3), b3p, g3[None, :], be3[None, :])

    idx = _knn(xyz1, xyz2)
    table = _build_table(points2, xyz2)
    grows = _sc_gather_rows(table, idx.reshape(1, B * N1 * K))

    wat = jnp.transpose(Wm1[:, 0:H])
    wbt = jnp.transpose(Wm1[:, H:H + IC])
    wct = jnp.transpose(Wm1[:, H + IC:H + IC + 3])
    out = _mlp(grows, p1, xyz1, wat, wbt, wct,
               bm1[None, :], jnp.transpose(Wm2), bm2[None, :],
               jnp.transpose(Wq1), jnp.transpose(Wq2), jnp.transpose(Wr),
               br[None, :])
    return jnp.transpose(out, (0, 2, 1))


# speedup vs baseline: 8.1264x; 1.3927x over previous
import functools

import jax
import jax.numpy as jnp
from jax import lax
from jax.experimental import pallas as pl
from jax.experimental.pallas import tpu as pltpu
from jax.experimental.pallas import tpu_sc as plsc

K = 32


def _dpad(c):
    return ((c + 3 + 127) // 128) * 128


def _lrelu(x):
    return jnp.where(x > 0, x, 0.1 * x)


def _p1conv_body(x_ref, w1, b1, g1, e1, w2, b2, g2, e2, w3, b3, g3, e3, o_ref):
    def layer(x, wt, bt, g, e):
        y = jnp.dot(x, wt[...], preferred_element_type=jnp.float32) + bt[...]
        m = jnp.mean(y, axis=0, keepdims=True)
        v = jnp.mean((y - m) ** 2, axis=0, keepdims=True)
        return _lrelu(g[...] * (y - m) / jnp.sqrt(v + 1e-5) + e[...])

    x = layer(x_ref[...], w1, b1, g1, e1)
    x = layer(x, w2, b2, g2, e2)
    o_ref[...] = layer(x, w3, b3, g3, e3)


def _p1conv(xp, w1t, b1p, g1, e1, w2t, b2p, g2, e2, w3t, b3p, g3, e3):
    P, H = xp.shape[0], w3t.shape[1]
    return pl.pallas_call(
        _p1conv_body,
        out_shape=jax.ShapeDtypeStruct((P, H), jnp.float32),
    )(xp, w1t, b1p, g1, e1, w2t, b2p, g2, e2, w3t, b3p, g3, e3)


_LSEL = 256
_RCAP = 6
_BIG = 1e30


def _knn_body(x1_ref, x2_ref, o_ref, dsc, avsc, aisc, n2):
    b = pl.program_id(0)
    L = _LSEL
    C = n2 // L
    x1t = jnp.swapaxes(x1_ref[0], 0, 1)
    x2 = x2_ref[0]
    n1 = x1t.shape[0]
    prod = jnp.dot(x1t, x2, preferred_element_type=jnp.float32)
    sq1 = jnp.sum(x1t * x1t, axis=1, keepdims=True)
    sq2 = jnp.sum(x2 * x2, axis=0, keepdims=True)
    dsc[...] = ((-2.0 * prod) + sq1) + sq2

    lane = lax.broadcasted_iota(jnp.int32, (n1, L), 1)
    col = lax.broadcasted_iota(jnp.int32, (n1, K), 1)
    avsc[...] = jnp.full(avsc.shape, _BIG, jnp.float32)
    aisc[...] = jnp.full(aisc.shape, 2.0 * _BIG, jnp.float32)

    lm0 = jnp.full((n1, L), _BIG, jnp.float32)
    am0 = jnp.zeros((n1, L), jnp.int32)
    for c in range(C):
        d = dsc[:, c * L:(c + 1) * L]
        upd = d < lm0
        am0 = jnp.where(upd, c, am0)
        lm0 = jnp.minimum(lm0, d)

    def round_body(carry):
        r, lm, am, done = carry
        avsc[r] = lm
        aisc[r] = (am * L + lane).astype(jnp.float32)
        nlm = jnp.full((n1, L), _BIG, jnp.float32)
        nam = jnp.zeros((n1, L), jnp.int32)
        for c in range(C):
            d = dsc[:, c * L:(c + 1) * L]
            d = jnp.where(am == c, _BIG, d)
            dsc[:, c * L:(c + 1) * L] = d
            upd = d < nlm
            nam = jnp.where(upd, c, nam)
            nlm = jnp.minimum(nlm, d)
        rrow = jnp.min(nlm, axis=1, keepdims=True)
        below = (avsc[...] < rrow[None, :, :]).astype(jnp.float32)
        cnt = jnp.sum(jnp.sum(below, axis=0), axis=1)
        return r + 1, nlm, nam, jnp.min(cnt) >= K

    def round_cond(carry):
        r, _, _, done = carry
        return jnp.logical_and(r < _RCAP, jnp.logical_not(done))

    _, _, _, ok = lax.while_loop(
        round_cond, round_body, (0, lm0, am0, False))

    def fast_fn(_):
        def ex(k, carry):
            av, acc = carry
            m = jnp.min(jnp.min(av, axis=0), axis=1, keepdims=True)
            tie = av == m[None, :, :]
            cand = jnp.where(tie, aisc[...], jnp.float32(4.0 * _BIG))
            i1 = jnp.min(jnp.min(cand, axis=0), axis=1, keepdims=True)
            sel = jnp.logical_and(tie, aisc[...] == i1[None, :, :])
            av = jnp.where(sel, _BIG, av)
            acc = jnp.where(col == k, i1.astype(jnp.int32), acc)
            return av, acc
        _, acc = lax.fori_loop(0, K, ex, (avsc[...],
                                          jnp.zeros((n1, K), jnp.int32)))
        return acc

    def slow_fn(_):
        dsc[...] = ((-2.0 * prod) + sq1) + sq2
        lane2 = lax.broadcasted_iota(jnp.int32, (n1, n2), 1)

        def ex(k, acc):
            d = dsc[...]
            idx = jnp.argmin(d, axis=1).astype(jnp.int32)[:, None]
            dsc[...] = jnp.where(lane2 == idx, _BIG, d)
            return jnp.where(col == k, idx, acc)
        return lax.fori_loop(0, K, ex, jnp.zeros((n1, K), jnp.int32))

    acc = lax.cond(ok, fast_fn, slow_fn, None)
    o_ref[0] = acc + b * n2


def _knn(xyz1, xyz2):
    B, _, N1 = xyz1.shape
    N2 = xyz2.shape[2]
    return pl.pallas_call(
        functools.partial(_knn_body, n2=N2),
        grid=(B,),
        in_specs=[
            pl.BlockSpec((1, 3, N1), lambda b: (b, 0, 0)),
            pl.BlockSpec((1, 3, N2), lambda b: (b, 0, 0)),
        ],
        out_specs=pl.BlockSpec((1, N1, K), lambda b: (b, 0, 0)),
        out_shape=jax.ShapeDtypeStruct((B, N1, K), jnp.int32),
        scratch_shapes=[
            pltpu.VMEM((N1, N2), jnp.float32),
            pltpu.VMEM((_RCAP, N1, _LSEL), jnp.float32),
            pltpu.VMEM((_RCAP, N1, _LSEL), jnp.float32),
        ],
    )(xyz1, xyz2)


def _table_body(p2_ref, x2_ref, o_ref, c, dpad):
    o_ref[:, 0:c] = jnp.swapaxes(p2_ref[0], 0, 1)
    o_ref[:, c:c + 3] = jnp.swapaxes(x2_ref[0], 0, 1)


def _build_table(points2, xyz2):
    B, C, N2 = points2.shape
    dpad = _dpad(C)
    J = 8
    W = N2 // J
    return pl.pallas_call(
        functools.partial(_table_body, c=C, dpad=dpad),
        grid=(B, J),
        in_specs=[
            pl.BlockSpec((1, C, W), lambda b, j: (b, 0, j)),
            pl.BlockSpec((1, 3, W), lambda b, j: (b, 0, j)),
        ],
        out_specs=pl.BlockSpec((W, dpad), lambda b, j: (b * J + j, 0)),
        out_shape=jax.ShapeDtypeStruct((B * N2, dpad), jnp.float32),
    )(points2, xyz2)


def _sc_gather_rows(table, idx2d):
    nsel = idx2d.shape[1]
    dpad = table.shape[1]
    W = 128
    mesh = plsc.VectorSubcoreMesh(core_axis_name="c", subcore_axis_name="s")

    @functools.partial(
        pl.kernel,
        out_type=jax.ShapeDtypeStruct((nsel, dpad), jnp.float32),
        mesh=mesh,
    )
    def k(table_hbm, i_hbm, o_hbm):
        def body(i_vmem, o_vmem):
            pltpu.sync_copy(table_hbm.at[i_vmem.at[0]], o_vmem)

        pltpu.emit_pipeline(
            body,
            grid=(nsel // W,),
            in_specs=[pl.BlockSpec((1, W), lambda i: (0, i))],
            out_specs=[pl.BlockSpec((W, dpad), lambda i: (i, 0))],
            core_axis_name=("c", "s"),
            dimension_semantics=(pltpu.PARALLEL,),
        )(i_hbm, o_hbm)

    return k(table, idx2d)


def _mlp_body(g_ref, p1_ref, x1_ref, wat, wbt, wct, bm1, w2t, bm2, wq1t,
              wq2t, wrt, br, o_ref, c):
    n1 = p1_ref.shape[0]
    h2dim = wat.shape[1]
    nbr = g_ref[:, c:c + 3]
    x1t = jnp.swapaxes(x1_ref[0], 0, 1)
    x1rep = jnp.broadcast_to(x1t[:, None, :], (n1, K, 3)).reshape(n1 * K, 3)
    dirn = nbr - x1rep

    z1 = jnp.dot(p1_ref[...], wat[...], preferred_element_type=jnp.float32)
    z1rep = jnp.broadcast_to(z1[:, None, :],
                             (n1, K, h2dim)).reshape(n1 * K, h2dim)
    zp2 = jnp.dot(g_ref[:, 0:c], wbt[...], preferred_element_type=jnp.float32)
    zdir = jnp.dot(dirn, wct[...], preferred_element_type=jnp.float32)
    h1 = _lrelu(zp2 + zdir + z1rep + bm1[...])
    h2 = _lrelu(jnp.dot(h1, w2t[...], preferred_element_type=jnp.float32)
                + bm2[...])
    pooled = jnp.max(h2.reshape(n1, K, h2dim), axis=1)
    q = _lrelu(jnp.dot(pooled, wq1t[...], preferred_element_type=jnp.float32))
    q = _lrelu(jnp.dot(q, wq2t[...], preferred_element_type=jnp.float32))
    o_ref[0] = jnp.dot(q, wrt[...], preferred_element_type=jnp.float32) + br[...]


def _mlp(grows, p1, xyz1, wat, wbt, wct, bm1, w2t, bm2, wq1t, wq2t, wrt, br):
    B, _, N1 = xyz1.shape
    H = wat.shape[0]
    C = wbt.shape[0]
    dpad = grows.shape[1]
    h2dim = wat.shape[1]
    return pl.pallas_call(
        functools.partial(_mlp_body, c=C),
        grid=(B,),
        in_specs=[
            pl.BlockSpec((N1 * K, dpad), lambda b: (b, 0)),
            pl.BlockSpec((N1, H), lambda b: (b, 0)),
            pl.BlockSpec((1, 3, N1), lambda b: (b, 0, 0)),
            pl.BlockSpec(wat.shape, lambda b: (0, 0)),
            pl.BlockSpec(wbt.shape, lambda b: (0, 0)),
            pl.BlockSpec(wct.shape, lambda b: (0, 0)),
            pl.BlockSpec((1, h2dim), lambda b: (0, 0)),
            pl.BlockSpec(w2t.shape, lambda b: (0, 0)),
            pl.BlockSpec((1, h2dim), lambda b: (0, 0)),
            pl.BlockSpec(wq1t.shape, lambda b: (0, 0)),
            pl.BlockSpec(wq2t.shape, lambda b: (0, 0)),
            pl.BlockSpec(wrt.shape, lambda b: (0, 0)),
            pl.BlockSpec((1, 3), lambda b: (0, 0)),
        ],
        out_specs=pl.BlockSpec((1, N1, 3), lambda b: (b, 0, 0)),
        out_shape=jax.ShapeDtypeStruct((B, N1, 3), jnp.float32),
    )(grows, p1, xyz1, wat, wbt, wct, bm1, w2t, bm2, wq1t, wq2t, wrt, br)


def kernel(xyz1, xyz2, points1, points2, W1, b1, g1, be1, W2, b2, g2, be2,
           W3, b3, g3, be3, Wm1, bm1, Wm2, bm2, Wq1, Wq2, Wr, br):
    B, _, N1 = xyz1.shape
    N2 = xyz2.shape[2]
    LC = points1.shape[1]
    IC = points2.shape[1]
    H = W1.shape[0]

    xp = jnp.transpose(points1, (0, 2, 1)).reshape(B * N1, LC)
    b1p = jnp.tile(jnp.transpose(b1), (B, 1))
    b2p = jnp.tile(jnp.transpose(b2), (B, 1))
    b3p = jnp.tile(jnp.transpose(b3), (B, 1))
    p1 = _p1conv(xp, jnp.transpose(W1), b1p, g1[None, :], be1[None, :],
                 jnp.transpose(W2), b2p, g2[None, :], be2[None, :],
                 jnp.transpose(W3), b3p, g3[None, :], be3[None, :])

    idx = _knn(xyz1, xyz2)
    table = _build_table(points2, xyz2)
    grows = _sc_gather_rows(table, idx.reshape(1, B * N1 * K))

    wat = jnp.transpose(Wm1[:, 0:H])
    wbt = jnp.transpose(Wm1[:, H:H + IC])
    wct = jnp.transpose(Wm1[:, H + IC:H + IC + 3])
    out = _mlp(grows, p1, xyz1, wat, wbt, wct,
               bm1[None, :], jnp.transpose(Wm2), bm2[None, :],
               jnp.transpose(Wq1), jnp.transpose(Wq2), jnp.transpose(Wr),
               br[None, :])
    return jnp.transpose(out, (0, 2, 1))
